# baseline (speedup 1.0000x reference)
import functools

import jax
import jax.numpy as jnp
from jax import lax
from jax.experimental import pallas as pl
from jax.experimental.pallas import tpu as pltpu
from jax.experimental.pallas import tpu_sc as plsc

N = 10000
E = 160000
D = 256
H = 128
NC = 2
NS = 16
EPS = 10240
CHUNK = 64
NCHUNKS = EPS // CHUNK
DCH = 128
DNCH = EPS // DCH
E_PAD = NS * EPS
N_PAD = 10240
RPS = N_PAD // NS
R = 1000
GRID = N // R

NBUF = 4
IDX_PH = 4
PCH = NCHUNKS // IDX_PH


@functools.lru_cache(maxsize=None)
def _get_mesh():
    return plsc.VectorSubcoreMesh(core_axis_name="c", subcore_axis_name="s",
                                  num_cores=NC, num_subcores=NS)


def _sc_agg_body(tbl, srcb, dstb, out, src_v, dst_v, gb, acc, *sems):
    sg = sems[:NBUF]
    ss = sems[NBUF:]
    c = lax.axis_index("c")
    s = lax.axis_index("s")

    def _zr(t, carry):
        gb[0, t // 8, pl.ds((t % 8) * 16, 16)] = jnp.zeros((16,), jnp.float32)
        return carry

    lax.fori_loop(0, CHUNK * 8, _zr, 0)
    for k in range(RPS // CHUNK):
        pltpu.sync_copy(gb.at[0], acc.at[pl.ds(s * RPS + k * CHUNK, CHUNK)])

    for ph in range(IDX_PH):
        pltpu.sync_copy(srcb.at[c, s, pl.ds(ph * PCH, PCH)], src_v)
        pltpu.sync_copy(dstb.at[s, pl.ds(ph * PCH, PCH)], dst_v)
        if ph == 0:
            plsc.subcore_barrier()

        for b in range(NBUF - 1):
            pltpu.async_copy(tbl.at[src_v.at[b]], gb.at[b], sg[b])

        def _step(jj, carry):
            for b in range(NBUF):
                j = jj * NBUF + b
                nxt = j + NBUF - 1
                nb = (b + NBUF - 1) % NBUF
                pltpu.make_async_copy(tbl.at[src_v.at[j]], gb.at[b],
                                      sg[b]).wait()

                @pl.when(jnp.logical_and(nxt < PCH, j >= 1))
                def _():
                    pltpu.make_async_copy(tbl.at[src_v.at[j]], gb.at[nb],
                                          ss[nb]).wait()

                @pl.when(nxt < PCH)
                def _():
                    pltpu.async_copy(tbl.at[src_v.at[nxt]], gb.at[nb],
                                     sg[nb])

                pltpu.async_copy(gb.at[b], acc.at[dst_v.at[j]], ss[b],
                                 add=True)
            return carry

        lax.fori_loop(0, PCH // NBUF, _step, 0)
        for b in range(NBUF):
            pltpu.make_async_copy(tbl.at[src_v.at[0]], gb.at[b],
                                  ss[b]).wait()

    plsc.subcore_barrier()
    pltpu.sync_copy(acc.at[pl.ds(s * RPS, RPS)],
                    out.at[c, pl.ds(s * RPS, RPS)])


def _sc_agg(tbl, srcb, dstb):
    return pl.kernel(
        _sc_agg_body,
        out_type=jax.ShapeDtypeStruct((NC, N_PAD, H), jnp.float32),
        mesh=_get_mesh(),
        scratch_types=[
            pltpu.VMEM((PCH, CHUNK), jnp.int32),
            pltpu.VMEM((PCH, CHUNK), jnp.int32),
            pltpu.VMEM((NBUF, CHUNK, H), jnp.float32),
            pltpu.VMEM_SHARED((N_PAD, H), jnp.float32),
        ] + [pltpu.SemaphoreType.DMA] * (2 * NBUF),
    )(tbl, srcb, dstb)


def _sc_deg_body(dstb, out, dst_v, obuf, acc):
    c = lax.axis_index("c")
    s = lax.axis_index("s")
    half = DNCH // NC

    def _fill(t, val):
        obuf[t // 8, pl.ds((t % 8) * 16, 16)] = jnp.full((16,), val,
                                                         jnp.float32)
        return val

    lax.fori_loop(0, DCH * 8, _fill, 0.0)
    for k in range(RPS // DCH):
        pltpu.sync_copy(obuf, acc.at[pl.ds(s * RPS + k * DCH, DCH)])
    lax.fori_loop(0, DCH * 8, _fill, 1.0)
    pltpu.sync_copy(dstb.at[s, pl.ds(c * half, half)], dst_v)
    plsc.subcore_barrier()

    def _step(j, carry):
        pltpu.sync_copy(obuf, acc.at[dst_v.at[j]], add=True)
        return carry

    lax.fori_loop(0, half, _step, 0)
    plsc.subcore_barrier()
    pltpu.sync_copy(acc.at[pl.ds(s * RPS, RPS)],
                    out.at[c, pl.ds(s * RPS, RPS)])


def _sc_deg(dstb):
    return pl.kernel(
        _sc_deg_body,
        out_type=jax.ShapeDtypeStruct((NC, N_PAD, H), jnp.float32),
        mesh=_get_mesh(),
        scratch_types=[
            pltpu.VMEM((DNCH // NC, DCH), jnp.int32),
            pltpu.VMEM((DCH, H), jnp.float32),
            pltpu.VMEM_SHARED((N_PAD, H), jnp.float32),
        ],
    )(dstb)


def _gelu(z):
    return 0.5 * z * (1.0 + lax.erf(z * 0.7071067811865476))


def _head_combine(a_ref, y_ref, d_ref, s_ref, act):
    dm = jnp.maximum(d_ref[0] + d_ref[1], 1.0)
    lo = a_ref[0] / dm + y_ref[0]
    hi = a_ref[1] / dm + y_ref[1]
    if s_ref is not None:
        lo = lo + s_ref[:, :H]
        hi = hi + s_ref[:, H:]
    if act:
        lo = _gelu(lo)
        hi = _gelu(hi)
    return jnp.concatenate([lo, hi], axis=1)


def _mm_body(*refs, fused, skip, keep_x):
    if fused:
        a_ref, y_ref, d_ref, *rest = refs
        s_ref = None
        if skip:
            s_ref, *rest = rest
        if keep_x:
            w_ref, b_ref, y_out, x_out = rest
        else:
            w_ref, b_ref, y_out = rest
        xb = _head_combine(a_ref, y_ref, d_ref, s_ref, act=True)
        if keep_x:
            x_out[...] = xb
    else:
        x_ref, w_ref, b_ref, y_out = refs
        xb = x_ref[...]
    y = jnp.dot(xb, w_ref[...], preferred_element_type=jnp.float32)
    y = y + b_ref[...]
    for k in range(4):
        y_out[k] = y[:, k * H:(k + 1) * H]


def _mm_call(x_or_parts, wcat, bcat, skip_x=None, keep_x=False):
    fused = isinstance(x_or_parts, tuple)
    if fused:
        agg2, y4p, degb = x_or_parts
        args = [agg2, y4p, degb]
        in_specs = [
            pl.BlockSpec((NC, R, H), lambda i: (0, i, 0)),
            pl.BlockSpec((2, R, H), lambda i: (1, i, 0)),
            pl.BlockSpec((NC, R, H), lambda i: (0, i, 0)),
        ]
        if skip_x is not None:
            args.append(skip_x)
            in_specs.append(pl.BlockSpec((R, D), lambda i: (i, 0)))
    else:
        args = [x_or_parts]
        in_specs = [pl.BlockSpec((R, D), lambda i: (i, 0))]
    args += [wcat, bcat]
    in_specs += [
        pl.BlockSpec((D, 2 * D), lambda i: (0, 0)),
        pl.BlockSpec((1, 2 * D), lambda i: (0, 0)),
    ]
    out_specs = [pl.BlockSpec((4, R, H), lambda i: (0, i, 0))]
    out_shape = [jax.ShapeDtypeStruct((4, N, H), jnp.float32)]
    if keep_x:
        out_specs.append(pl.BlockSpec((R, D), lambda i: (i, 0)))
        out_shape.append(jax.ShapeDtypeStruct((N, D), jnp.float32))
    res = pl.pallas_call(
        functools.partial(_mm_body, fused=fused, skip=skip_x is not None,
                          keep_x=keep_x),
        grid=(GRID,),
        in_specs=in_specs,
        out_specs=out_specs,
        out_shape=out_shape,
    )(*args)
    return res


def _comb_body(a_ref, y_ref, d_ref, o_ref):
    o_ref[...] = _head_combine(a_ref, y_ref, d_ref, None, act=False)


def _comb_call(agg2, y4, degb):
    return pl.pallas_call(
        _comb_body,
        grid=(GRID,),
        in_specs=[
            pl.BlockSpec((NC, R, H), lambda i: (0, i, 0)),
            pl.BlockSpec((2, R, H), lambda i: (1, i, 0)),
            pl.BlockSpec((NC, R, H), lambda i: (0, i, 0)),
        ],
        out_specs=pl.BlockSpec((R, D), lambda i: (i, 0)),
        out_shape=jax.ShapeDtypeStruct((N, D), jnp.float32),
    )(agg2, y4, degb)


def kernel(x, edge_index, Wl0, bl0, Wr0, Wl1, bl1, Wr1, Wl2, bl2, Wr2,
           Wl3, bl3, Wr3):
    src = edge_index[0].astype(jnp.int32)
    dst = edge_index[1].astype(jnp.int32)
    perm = jnp.argsort(src)
    src = src[perm]
    dst = dst[perm]
    pad = E_PAD - E
    srcp = jnp.concatenate([src, jnp.zeros((pad,), jnp.int32)])
    dstp = jnp.concatenate([dst, jnp.full((pad,), N, jnp.int32)])
    dstb = dstp.reshape(NS, NCHUNKS, CHUNK)
    dstb_d = dstp.reshape(NS, DNCH, DCH)
    s3 = srcp.reshape(1, NS, NCHUNKS, CHUNK)
    srcb = jnp.concatenate([s3, s3 + N], axis=0)

    degb = _sc_deg(dstb_d)

    def _wb(Wl, bl, Wr):
        wcat = jnp.concatenate([Wl.T, Wr.T], axis=1)
        bcat = jnp.concatenate([jnp.zeros((D,), jnp.float32), bl])
        return wcat, bcat.reshape(1, 2 * D)

    def _agg(y4):
        return _sc_agg(y4.reshape(4 * N, H), srcb, dstb)

    (y0,) = _mm_call(x, *_wb(Wl0, bl0, Wr0))
    a0 = _agg(y0)
    y1, x1 = _mm_call((a0, y0, degb), *_wb(Wl1, bl1, Wr1), keep_x=True)
    a1 = _agg(y1)
    (y2,) = _mm_call((a1, y1, degb), *_wb(Wl2, bl2, Wr2))
    a2 = _agg(y2)
    (y3,) = _mm_call((a2, y2, degb), *_wb(Wl3, bl3, Wr3), skip_x=x1)
    a3 = _agg(y3)
    return _comb_call(a3, y3, degb)


# speedup vs baseline: 1.2459x; 1.2459x over previous
import functools

import jax
import jax.numpy as jnp
from jax import lax
from jax.experimental import pallas as pl
from jax.experimental.pallas import tpu as pltpu
from jax.experimental.pallas import tpu_sc as plsc

N = 10000
E = 160000
D = 256
H = 128
NC = 2
NS = 16
EPS = 10240
CHUNK = 64
NCHUNKS = EPS // CHUNK
DCH = 128
DNCH = EPS // DCH
E_PAD = NS * EPS
N_PAD = 10240
RPS = N_PAD // NS
R = 1000
GRID = N // R

NBUF = 4
IDX_PH = 4
PCH = NCHUNKS // IDX_PH


@functools.lru_cache(maxsize=None)
def _get_mesh():
    return plsc.VectorSubcoreMesh(core_axis_name="c", subcore_axis_name="s",
                                  num_cores=NC, num_subcores=NS)


def _sc_agg_body(tbl, srcb, dstb, out, src_v, dst_v, gb, acc, *sems):
    sg = sems[:NBUF]
    ss = sems[NBUF:]
    c = lax.axis_index("c")
    s = lax.axis_index("s")

    def _zr(t, carry):
        gb[0, t // 8, pl.ds((t % 8) * 16, 16)] = jnp.zeros((16,), jnp.float32)
        return carry

    lax.fori_loop(0, CHUNK * 8, _zr, 0)
    for k in range(RPS // CHUNK):
        pltpu.sync_copy(gb.at[0], acc.at[pl.ds(s * RPS + k * CHUNK, CHUNK)])

    for ph in range(IDX_PH):
        pltpu.sync_copy(srcb.at[c, s, pl.ds(ph * PCH, PCH)], src_v)
        pltpu.sync_copy(dstb.at[s, pl.ds(ph * PCH, PCH)], dst_v)
        if ph == 0:
            plsc.subcore_barrier()

        for b in range(NBUF - 1):
            pltpu.async_copy(tbl.at[src_v.at[b]], gb.at[b], sg[b])

        def _step(jj, carry):
            for b in range(NBUF):
                j = jj * NBUF + b
                nxt = j + NBUF - 1
                nb = (b + NBUF - 1) % NBUF
                pltpu.make_async_copy(tbl.at[src_v.at[j]], gb.at[b],
                                      sg[b]).wait()

                @pl.when(jnp.logical_and(nxt < PCH, j >= 1))
                def _():
                    pltpu.make_async_copy(tbl.at[src_v.at[j]], gb.at[nb],
                                          ss[nb]).wait()

                @pl.when(nxt < PCH)
                def _():
                    pltpu.async_copy(tbl.at[src_v.at[nxt]], gb.at[nb],
                                     sg[nb])

                pltpu.async_copy(gb.at[b], acc.at[dst_v.at[j]], ss[b],
                                 add=True)
            return carry

        lax.fori_loop(0, PCH // NBUF, _step, 0)
        for b in range(NBUF):
            pltpu.make_async_copy(tbl.at[src_v.at[0]], gb.at[b],
                                  ss[b]).wait()

    plsc.subcore_barrier()
    pltpu.sync_copy(acc.at[pl.ds(s * RPS, RPS)],
                    out.at[c, pl.ds(s * RPS, RPS)])


def _sc_agg(tbl, srcb, dstb):
    return pl.kernel(
        _sc_agg_body,
        out_type=jax.ShapeDtypeStruct((NC, N_PAD, H), jnp.float32),
        mesh=_get_mesh(),
        scratch_types=[
            pltpu.VMEM((PCH, CHUNK), jnp.int32),
            pltpu.VMEM((PCH, CHUNK), jnp.int32),
            pltpu.VMEM((NBUF, CHUNK, H), jnp.float32),
            pltpu.VMEM_SHARED((N_PAD, H), jnp.float32),
        ] + [pltpu.SemaphoreType.DMA] * (2 * NBUF),
    )(tbl, srcb, dstb)


def _sc_deg_body(dstb, out, dst_v, obuf, acc):
    c = lax.axis_index("c")
    s = lax.axis_index("s")
    half = DNCH // NC

    def _fill(t, val):
        obuf[t // 8, pl.ds((t % 8) * 16, 16)] = jnp.full((16,), val,
                                                         jnp.float32)
        return val

    lax.fori_loop(0, DCH * 8, _fill, 0.0)
    for k in range(RPS // DCH):
        pltpu.sync_copy(obuf, acc.at[pl.ds(s * RPS + k * DCH, DCH)])
    lax.fori_loop(0, DCH * 8, _fill, 1.0)
    pltpu.sync_copy(dstb.at[s, pl.ds(c * half, half)], dst_v)
    plsc.subcore_barrier()

    def _step(j, carry):
        pltpu.sync_copy(obuf, acc.at[dst_v.at[j]], add=True)
        return carry

    lax.fori_loop(0, half, _step, 0)
    plsc.subcore_barrier()
    pltpu.sync_copy(acc.at[pl.ds(s * RPS, RPS)],
                    out.at[c, pl.ds(s * RPS, RPS)])


def _sc_deg(dstb):
    return pl.kernel(
        _sc_deg_body,
        out_type=jax.ShapeDtypeStruct((NC, N_PAD, H), jnp.float32),
        mesh=_get_mesh(),
        scratch_types=[
            pltpu.VMEM((DNCH // NC, DCH), jnp.int32),
            pltpu.VMEM((DCH, H), jnp.float32),
            pltpu.VMEM_SHARED((N_PAD, H), jnp.float32),
        ],
    )(dstb)


def _gelu(z):
    return 0.5 * z * (1.0 + lax.erf(z * 0.7071067811865476))


def _head_combine(a_ref, y_ref, d_ref, s_ref, act):
    dm = jnp.maximum(d_ref[0] + d_ref[1], 1.0)
    lo = a_ref[0] / dm + y_ref[0]
    hi = a_ref[1] / dm + y_ref[1]
    if s_ref is not None:
        lo = lo + s_ref[:, :H]
        hi = hi + s_ref[:, H:]
    if act:
        lo = _gelu(lo)
        hi = _gelu(hi)
    return jnp.concatenate([lo, hi], axis=1)


def _mm_body(*refs, fused, skip, keep_x):
    if fused:
        a_ref, y_ref, d_ref, *rest = refs
        s_ref = None
        if skip:
            s_ref, *rest = rest
        if keep_x:
            w_ref, b_ref, y_out, x_out = rest
        else:
            w_ref, b_ref, y_out = rest
        xb = _head_combine(a_ref, y_ref, d_ref, s_ref, act=True)
        if keep_x:
            x_out[...] = xb
    else:
        x_ref, w_ref, b_ref, y_out = refs
        xb = x_ref[...]
    y = jnp.dot(xb, w_ref[...], preferred_element_type=jnp.float32)
    y = y + b_ref[...]
    for k in range(4):
        y_out[k] = y[:, k * H:(k + 1) * H]


def _mm_call(x_or_parts, wcat, bcat, skip_x=None, keep_x=False):
    fused = isinstance(x_or_parts, tuple)
    if fused:
        agg2, y4p, degb = x_or_parts
        args = [agg2, y4p, degb]
        in_specs = [
            pl.BlockSpec((NC, R, H), lambda i: (0, i, 0)),
            pl.BlockSpec((2, R, H), lambda i: (1, i, 0)),
            pl.BlockSpec((NC, R, H), lambda i: (0, i, 0)),
        ]
        if skip_x is not None:
            args.append(skip_x)
            in_specs.append(pl.BlockSpec((R, D), lambda i: (i, 0)))
    else:
        args = [x_or_parts]
        in_specs = [pl.BlockSpec((R, D), lambda i: (i, 0))]
    args += [wcat, bcat]
    in_specs += [
        pl.BlockSpec((D, 2 * D), lambda i: (0, 0)),
        pl.BlockSpec((1, 2 * D), lambda i: (0, 0)),
    ]
    out_specs = [pl.BlockSpec((4, R, H), lambda i: (0, i, 0))]
    out_shape = [jax.ShapeDtypeStruct((4, N, H), jnp.float32)]
    if keep_x:
        out_specs.append(pl.BlockSpec((R, D), lambda i: (i, 0)))
        out_shape.append(jax.ShapeDtypeStruct((N, D), jnp.float32))
    res = pl.pallas_call(
        functools.partial(_mm_body, fused=fused, skip=skip_x is not None,
                          keep_x=keep_x),
        grid=(GRID,),
        in_specs=in_specs,
        out_specs=out_specs,
        out_shape=out_shape,
    )(*args)
    return res


def _comb_body(a_ref, y_ref, d_ref, o_ref):
    o_ref[...] = _head_combine(a_ref, y_ref, d_ref, None, act=False)


def _comb_call(agg2, y4, degb):
    return pl.pallas_call(
        _comb_body,
        grid=(GRID,),
        in_specs=[
            pl.BlockSpec((NC, R, H), lambda i: (0, i, 0)),
            pl.BlockSpec((2, R, H), lambda i: (1, i, 0)),
            pl.BlockSpec((NC, R, H), lambda i: (0, i, 0)),
        ],
        out_specs=pl.BlockSpec((R, D), lambda i: (i, 0)),
        out_shape=jax.ShapeDtypeStruct((N, D), jnp.float32),
    )(agg2, y4, degb)


def kernel(x, edge_index, Wl0, bl0, Wr0, Wl1, bl1, Wr1, Wl2, bl2, Wr2,
           Wl3, bl3, Wr3):
    src = edge_index[0].astype(jnp.int32)
    dst = edge_index[1].astype(jnp.int32)
    pad = E_PAD - E
    srcp = jnp.concatenate([src, jnp.zeros((pad,), jnp.int32)])
    dstp = jnp.concatenate([dst, jnp.full((pad,), N, jnp.int32)])
    dstb = dstp.reshape(NS, NCHUNKS, CHUNK)
    dstb_d = dstp.reshape(NS, DNCH, DCH)
    s3 = srcp.reshape(1, NS, NCHUNKS, CHUNK)
    srcb = jnp.concatenate([s3, s3 + N], axis=0)

    degb = _sc_deg(dstb_d)

    def _wb(Wl, bl, Wr):
        wcat = jnp.concatenate([Wl.T, Wr.T], axis=1)
        bcat = jnp.concatenate([jnp.zeros((D,), jnp.float32), bl])
        return wcat, bcat.reshape(1, 2 * D)

    def _agg(y4):
        return _sc_agg(y4.reshape(4 * N, H), srcb, dstb)

    (y0,) = _mm_call(x, *_wb(Wl0, bl0, Wr0))
    a0 = _agg(y0)
    y1, x1 = _mm_call((a0, y0, degb), *_wb(Wl1, bl1, Wr1), keep_x=True)
    a1 = _agg(y1)
    (y2,) = _mm_call((a1, y1, degb), *_wb(Wl2, bl2, Wr2))
    a2 = _agg(y2)
    (y3,) = _mm_call((a2, y2, degb), *_wb(Wl3, bl3, Wr3), skip_x=x1)
    a3 = _agg(y3)
    return _comb_call(a3, y3, degb)
